# baseline (speedup 1.0000x reference)
import functools

import jax
import jax.numpy as jnp
from jax import lax
from jax.experimental import pallas as pl
from jax.experimental.pallas import tpu as pltpu
from jax.experimental.pallas import tpu_sc as plsc

N_NODES = 10000
N_EDGES = 320000
D = 128

NC = 2
NS = 16
NW = NC * NS

CHUNK = 128
NBUF = 3
N_CHUNKS = 81
PER_TILE = N_CHUNKS * CHUNK
E_PAD = PER_TILE * NW

TILE_ROWS = 632
ACC_ROWS = TILE_ROWS * NS

_MM_BLK = 2000


def _mm_body(x_ref, w_ref, ws_ref, xw_ref, xself_ref):
    xb = x_ref[...]
    xw_ref[...] = jnp.dot(xb, w_ref[...], preferred_element_type=jnp.float32)
    xself_ref[...] = jnp.dot(xb, ws_ref[...], preferred_element_type=jnp.float32)


def _matmul2(x, w, ws):
    grid = (N_NODES // _MM_BLK,)
    return pl.pallas_call(
        _mm_body,
        grid=grid,
        in_specs=[
            pl.BlockSpec((_MM_BLK, D), lambda i: (i, 0)),
            pl.BlockSpec((D, D), lambda i: (0, 0)),
            pl.BlockSpec((D, D), lambda i: (0, 0)),
        ],
        out_specs=[
            pl.BlockSpec((_MM_BLK, D), lambda i: (i, 0)),
            pl.BlockSpec((_MM_BLK, D), lambda i: (i, 0)),
        ],
        out_shape=[
            jax.ShapeDtypeStruct((N_NODES, D), jnp.float32),
            jax.ShapeDtypeStruct((N_NODES, D), jnp.float32),
        ],
    )(x, w, ws)


_sc_mesh = plsc.VectorSubcoreMesh(core_axis_name="c", subcore_axis_name="s")


@functools.partial(
    pl.kernel,
    mesh=_sc_mesh,
    out_type=jax.ShapeDtypeStruct((NC, ACC_ROWS, D), jnp.float32),
    scratch_types=[
        pltpu.VMEM_SHARED((ACC_ROWS, D), jnp.float32),
        pltpu.VMEM((NBUF, CHUNK), jnp.int32),
        pltpu.VMEM((NBUF, CHUNK), jnp.int32),
        pltpu.VMEM((NBUF, CHUNK, D), jnp.float32),
        pltpu.SemaphoreType.DMA,
        pltpu.SemaphoreType.DMA,
        pltpu.SemaphoreType.DMA,
        pltpu.SemaphoreType.DMA,
        pltpu.SemaphoreType.DMA,
        pltpu.SemaphoreType.DMA,
        pltpu.SemaphoreType.DMA,
        pltpu.SemaphoreType.DMA,
    ],
)
def _sc_scatter(xw_hbm, src_hbm, dst_hbm, z_hbm, out_hbm,
                acc, srcs, dsts, rows,
                g0, g1, g2, s0, s1, s2, lsrc, ldst):
    c = lax.axis_index("c")
    s = lax.axis_index("s")
    w = c * NS + s
    gsem = (g0, g1, g2)
    ssem = (s0, s1, s2)

    pltpu.sync_copy(z_hbm, acc.at[pl.ds(s * TILE_ROWS, TILE_ROWS)])
    plsc.subcore_barrier()

    base = w * (N_CHUNKS * CHUNK)

    def load_src(j, b):
        pltpu.async_copy(src_hbm.at[pl.ds(base + j * CHUNK, CHUNK)],
                         srcs.at[b], lsrc)

    def wait_src(b):
        pltpu.make_async_copy(src_hbm.at[pl.ds(0, CHUNK)], srcs.at[b],
                              lsrc).wait()

    def load_dst(j, b):
        pltpu.async_copy(dst_hbm.at[pl.ds(base + j * CHUNK, CHUNK)],
                         dsts.at[b], ldst)

    def wait_dst(b):
        pltpu.make_async_copy(dst_hbm.at[pl.ds(0, CHUNK)], dsts.at[b],
                              ldst).wait()

    def gather(b):
        pltpu.async_copy(xw_hbm.at[srcs.at[b]], rows.at[b], gsem[b])

    def wait_gather(b):
        pltpu.make_async_copy(xw_hbm.at[srcs.at[b]], rows.at[b], gsem[b]).wait()

    def scatter(b):
        pltpu.async_copy(rows.at[b], acc.at[dsts.at[b]], sem=ssem[b], add=True)

    def wait_scatter(b):
        pltpu.make_async_copy(rows.at[b], acc.at[dsts.at[b]], ssem[b]).wait()

    load_src(0, 0)
    wait_src(0)
    gather(0)
    load_src(1, 1)
    wait_src(1)
    gather(1)
    load_src(2, 2)
    load_dst(0, 0)

    wait_gather(0)
    wait_dst(0)
    scatter(0)
    load_dst(1, 1)
    wait_src(2)
    gather(2)
    load_src(3, 0)

    def steady(gidx, carry):
        for u in range(3):
            j = 1 + gidx * 3 + u
            b = (1 + u) % 3
            bm1 = u
            bp1 = (2 + u) % 3
            bp2 = u
            wait_gather(b)
            wait_dst(b)
            scatter(b)
            wait_scatter(bm1)
            load_dst(j + 1, bp1)
            wait_src(bp2)
            gather(bp2)

            @pl.when(j <= N_CHUNKS - 4)
            def _():
                load_src(j + 3, b)
        return carry

    lax.fori_loop(0, (N_CHUNKS - 3) // 3, steady, 0)

    wait_gather(1)
    wait_dst(1)
    scatter(1)
    wait_scatter(0)
    load_dst(N_CHUNKS - 1, 2)
    wait_gather(2)
    wait_dst(2)
    scatter(2)
    wait_scatter(1)
    wait_scatter(2)

    plsc.subcore_barrier()

    pltpu.sync_copy(acc.at[pl.ds(s * TILE_ROWS, TILE_ROWS)],
                    out_hbm.at[c, pl.ds(s * TILE_ROWS, TILE_ROWS)])


def _add_body(p_ref, s_ref, o_ref):
    o_ref[...] = p_ref[0] + p_ref[1] + s_ref[...]


def _combine(partial, xself):
    grid = (N_NODES // _MM_BLK,)
    return pl.pallas_call(
        _add_body,
        grid=grid,
        in_specs=[
            pl.BlockSpec((NC, _MM_BLK, D), lambda i: (0, i, 0)),
            pl.BlockSpec((_MM_BLK, D), lambda i: (i, 0)),
        ],
        out_specs=pl.BlockSpec((_MM_BLK, D), lambda i: (i, 0)),
        out_shape=jax.ShapeDtypeStruct((N_NODES, D), jnp.float32),
    )(partial, xself)


def kernel(x, edge_index, weight, weight_self):
    xw, xself = _matmul2(x, weight, weight_self)
    src = edge_index[0]
    dst = edge_index[1]
    pad = E_PAD - N_EDGES
    src_p = jnp.concatenate([src, jnp.zeros((pad,), jnp.int32)])
    dst_p = jnp.concatenate([dst, jnp.full((pad,), N_NODES, jnp.int32)])
    z_rows = jnp.zeros((TILE_ROWS, D), jnp.float32)
    partial = _sc_scatter(xw, src_p, dst_p, z_rows)
    return _combine(partial, xself)


# speedup vs baseline: 1.2478x; 1.2478x over previous
import functools

import jax
import jax.numpy as jnp
from jax import lax
from jax.experimental import pallas as pl
from jax.experimental.pallas import tpu as pltpu
from jax.experimental.pallas import tpu_sc as plsc

N_NODES = 10000
N_EDGES = 320000
D = 128

NC = 2
NS = 16
NW = NC * NS

CHUNK = 128
N_CHUNKS = 80
PER_TILE = N_CHUNKS * CHUNK
E_PAD = PER_TILE * NW

TILE_ROWS = 632
ACC_ROWS = TILE_ROWS * NS

_MM_BLK = 2000


def _mm_body(x_ref, w_ref, ws_ref, xw_ref, xself_ref):
    xb = x_ref[...]
    xw_ref[...] = jnp.dot(xb, w_ref[...], preferred_element_type=jnp.float32)
    xself_ref[...] = jnp.dot(xb, ws_ref[...], preferred_element_type=jnp.float32)


def _matmul2(x, w, ws):
    grid = (N_NODES // _MM_BLK,)
    return pl.pallas_call(
        _mm_body,
        grid=grid,
        in_specs=[
            pl.BlockSpec((_MM_BLK, D), lambda i: (i, 0)),
            pl.BlockSpec((D, D), lambda i: (0, 0)),
            pl.BlockSpec((D, D), lambda i: (0, 0)),
        ],
        out_specs=[
            pl.BlockSpec((_MM_BLK, D), lambda i: (i, 0)),
            pl.BlockSpec((_MM_BLK, D), lambda i: (i, 0)),
        ],
        out_shape=[
            jax.ShapeDtypeStruct((N_NODES, D), jnp.float32),
            jax.ShapeDtypeStruct((N_NODES, D), jnp.float32),
        ],
    )(x, w, ws)


_sc_mesh = plsc.VectorSubcoreMesh(core_axis_name="c", subcore_axis_name="s")


@functools.partial(
    pl.kernel,
    mesh=_sc_mesh,
    out_type=jax.ShapeDtypeStruct((NC, ACC_ROWS, D), jnp.float32),
    scratch_types=[
        pltpu.VMEM_SHARED((ACC_ROWS, D), jnp.float32),
        pltpu.VMEM((N_CHUNKS, CHUNK), jnp.int32),
        pltpu.VMEM((N_CHUNKS, CHUNK), jnp.int32),
        pltpu.VMEM((CHUNK, D), jnp.float32),
        pltpu.SemaphoreType.DMA,
    ],
)
def _sc_scatter(xw_hbm, src_hbm, dst_hbm, z_hbm, out_hbm,
                acc, srcs, dsts, rows, gsem):
    c = lax.axis_index("c")
    s = lax.axis_index("s")
    w = c * NS + s

    pltpu.sync_copy(z_hbm, acc.at[pl.ds(s * TILE_ROWS, TILE_ROWS)])
    pltpu.sync_copy(src_hbm.at[w], srcs)
    pltpu.sync_copy(dst_hbm.at[w], dsts)
    plsc.subcore_barrier()

    def chunk_body(j, carry):
        pltpu.async_copy(xw_hbm.at[srcs.at[j]], rows, gsem).wait()
        pltpu.sync_copy(rows, acc.at[dsts.at[j]], add=True)
        return carry

    lax.fori_loop(0, N_CHUNKS, chunk_body, 0)
    plsc.subcore_barrier()

    plsc.subcore_barrier()

    pltpu.sync_copy(acc.at[pl.ds(s * TILE_ROWS, TILE_ROWS)],
                    out_hbm.at[c, pl.ds(s * TILE_ROWS, TILE_ROWS)])


def _add_body(p_ref, s_ref, o_ref):
    o_ref[...] = p_ref[0] + p_ref[1] + s_ref[...]


def _combine(partial, xself):
    grid = (N_NODES // _MM_BLK,)
    return pl.pallas_call(
        _add_body,
        grid=grid,
        in_specs=[
            pl.BlockSpec((NC, _MM_BLK, D), lambda i: (0, i, 0)),
            pl.BlockSpec((_MM_BLK, D), lambda i: (i, 0)),
        ],
        out_specs=pl.BlockSpec((_MM_BLK, D), lambda i: (i, 0)),
        out_shape=jax.ShapeDtypeStruct((N_NODES, D), jnp.float32),
    )(partial, xself)


def kernel(x, edge_index, weight, weight_self):
    xw, xself = _matmul2(x, weight, weight_self)
    src = edge_index[0]
    dst = edge_index[1]
    pad = E_PAD - N_EDGES
    src_p = jnp.concatenate([src, jnp.zeros((pad,), jnp.int32)])
    dst_p = jnp.concatenate([dst, jnp.full((pad,), N_NODES, jnp.int32)])
    src3 = src_p.reshape(NW, N_CHUNKS, CHUNK)
    dst3 = dst_p.reshape(NW, N_CHUNKS, CHUNK)
    z_rows = jnp.zeros((TILE_ROWS, D), jnp.float32)
    partial = _sc_scatter(xw, src3, dst3, z_rows)
    return _combine(partial, xself)
